# Initial kernel scaffold; baseline (speedup 1.0000x reference)
import jax
import jax.numpy as jnp
from jax.experimental import pallas as pl


def kernel(x, edge_index, Wl1, bl1, Wr1, Wl2, bl2, Wr2):
    raise NotImplementedError("write your pallas kernel here")


# speedup vs baseline: 3.2680x; 3.2680x over previous
import jax
import jax.numpy as jnp
from jax import lax
from jax.experimental import pallas as pl
from jax.experimental.pallas import tpu as pltpu
from jax.experimental.pallas import tpu_sc as plsc

N = 10000
E = 320000
D = 128
NC = 2
NS = 16
NW = NC * NS
NP = 10240
B = 128
NB = 80
EP = NW * NB * B
RPT = NP // NS
DW = 16


def _sc_agg():
    mesh = plsc.VectorSubcoreMesh(core_axis_name="c", subcore_axis_name="s")
    out_type = jax.ShapeDtypeStruct((NC, NP, D), jnp.float32)
    scratch = [
        pltpu.VMEM((NB, B), jnp.int32),
        pltpu.VMEM((NB, B), jnp.int32),
        pltpu.VMEM((B, D), jnp.float32),
        pltpu.VMEM_SHARED((NP, D), jnp.float32),
    ]

    def body(x_hbm, srcs, dsts, z_row, out_agg, srcv, dstv, rows, acc):
        c = lax.axis_index("c")
        s = lax.axis_index("s")
        wid = s * NC + c
        pltpu.sync_copy(z_row, acc.at[pl.ds(s * RPT, RPT)])
        pltpu.sync_copy(srcs.at[wid], srcv)
        pltpu.sync_copy(dsts.at[wid], dstv)
        plsc.subcore_barrier()

        def step(j, carry):
            pltpu.sync_copy(x_hbm.at[srcv.at[j]], rows)
            pltpu.sync_copy(rows, acc.at[dstv.at[j]], add=True)
            return carry

        lax.fori_loop(0, NB, step, 0)
        plsc.subcore_barrier()
        pltpu.sync_copy(acc.at[pl.ds(s * RPT, RPT)],
                        out_agg.at[c, pl.ds(s * RPT, RPT)])

    return pl.kernel(body, out_type=out_type, mesh=mesh, scratch_types=scratch)


def _sc_deg():
    mesh = plsc.VectorSubcoreMesh(core_axis_name="c", subcore_axis_name="s")
    out_type = jax.ShapeDtypeStruct((NC, NP), jnp.float32)
    scratch = [
        pltpu.VMEM((NB, B), jnp.int32),
        pltpu.VMEM((B,), jnp.float32),
        pltpu.VMEM_SHARED((NP,), jnp.float32),
    ]

    def body(dsts, z_deg, ones_h, out_deg, dstv, onesv, dacc):
        c = lax.axis_index("c")
        s = lax.axis_index("s")
        wid = s * NC + c
        pltpu.sync_copy(z_deg, dacc.at[pl.ds(s * RPT, RPT)])
        pltpu.sync_copy(ones_h, onesv)
        pltpu.sync_copy(dsts.at[wid], dstv)
        plsc.subcore_barrier()

        def step(j, carry):
            pltpu.sync_copy(onesv, dacc.at[dstv.at[j]], add=True)
            return carry

        lax.fori_loop(0, NB, step, 0)
        plsc.subcore_barrier()
        pltpu.sync_copy(dacc.at[pl.ds(s * RPT, RPT)],
                        out_deg.at[c, pl.ds(s * RPT, RPT)])

    return pl.kernel(body, out_type=out_type, mesh=mesh, scratch_types=scratch)


_AGG = _sc_agg()
_DEG = _sc_deg()


def _dense(aggp, degp, xin, WlT, bl2d, WrT, relu):
    R = 1000

    def body(aggp_ref, degp_ref, x_ref, wl_ref, bl_ref, wr_ref, out_ref):
        agg = aggp_ref[0] + aggp_ref[1]
        deg = degp_ref[0] + degp_ref[1]
        mean = agg / jnp.maximum(deg, 1.0)
        h = (jnp.dot(mean, wl_ref[...], preferred_element_type=jnp.float32)
             + bl_ref[...]
             + jnp.dot(x_ref[...], wr_ref[...], preferred_element_type=jnp.float32))
        if relu:
            h = jnp.maximum(h, 0.0)
        out_ref[...] = h

    return pl.pallas_call(
        body,
        grid=(N // R,),
        in_specs=[
            pl.BlockSpec((NC, R, D), lambda i: (0, i, 0)),
            pl.BlockSpec((NC, R, 1), lambda i: (0, i, 0)),
            pl.BlockSpec((R, D), lambda i: (i, 0)),
            pl.BlockSpec((D, D), lambda i: (0, 0)),
            pl.BlockSpec((1, D), lambda i: (0, 0)),
            pl.BlockSpec((D, D), lambda i: (0, 0)),
        ],
        out_specs=pl.BlockSpec((R, D), lambda i: (i, 0)),
        out_shape=jax.ShapeDtypeStruct((N, D), jnp.float32),
    )(aggp, degp, xin, WlT, bl2d, WrT)


def kernel(x, edge_index, Wl1, bl1, Wr1, Wl2, bl2, Wr2):
    npad = EP - E
    src = jnp.concatenate(
        [edge_index[0].astype(jnp.int32), jnp.zeros((npad,), jnp.int32)]
    ).reshape(NW, NB, B)
    dst = jnp.concatenate(
        [edge_index[1].astype(jnp.int32), jnp.full((npad,), N, jnp.int32)]
    ).reshape(NW, NB, B)
    z_row = jnp.zeros((RPT, D), jnp.float32)
    z_deg = jnp.zeros((RPT,), jnp.float32)
    ones_h = jnp.ones((B,), jnp.float32)

    degp = _DEG(dst, z_deg, ones_h).reshape(NC, NP, 1)
    aggp1 = _AGG(x, src, dst, z_row)
    h = _dense(aggp1, degp, x, Wl1.T, bl1.reshape(1, D), Wr1.T, relu=True)
    aggp2 = _AGG(h, src, dst, z_row)
    out = _dense(aggp2, degp, h, Wl2.T, bl2.reshape(1, D), Wr2.T, relu=False)
    return out


# speedup vs baseline: 3.2943x; 1.0081x over previous
import jax
import jax.numpy as jnp
from jax import lax
from jax.experimental import pallas as pl
from jax.experimental.pallas import tpu as pltpu
from jax.experimental.pallas import tpu_sc as plsc

N = 10000
E = 320000
D = 128
NC = 2
NS = 16
NW = NC * NS
NP = 10240
EPW = 10240
EP = NW * EPW
BE = 160
NBA = EPW // BE
B = 128
NB = EPW // B
RPT = NP // NS
DW = 16


def _sc_agg():
    mesh = plsc.VectorSubcoreMesh(core_axis_name="c", subcore_axis_name="s")
    out_type = jax.ShapeDtypeStruct((NC, NP, D), jnp.float32)
    scratch = [
        pltpu.VMEM((EPW,), jnp.int32),
        pltpu.VMEM((EPW,), jnp.int32),
        pltpu.VMEM((BE, D), jnp.float32),
        pltpu.VMEM_SHARED((NP, D), jnp.float32),
    ]

    def body(x_hbm, srcs, dsts, z_row, out_agg, srcv, dstv, rows, acc):
        c = lax.axis_index("c")
        s = lax.axis_index("s")
        wid = c * NS + s
        pltpu.sync_copy(z_row, acc.at[pl.ds(s * RPT, RPT)])
        pltpu.sync_copy(srcs.at[pl.ds(wid * EPW, EPW)], srcv)
        pltpu.sync_copy(dsts.at[pl.ds(wid * EPW, EPW)], dstv)
        plsc.subcore_barrier()

        def step(j, carry):
            idx = pl.ds(j * BE, BE)
            pltpu.sync_copy(x_hbm.at[srcv.at[idx]], rows)
            pltpu.sync_copy(rows, acc.at[dstv.at[idx]], add=True)
            return carry

        lax.fori_loop(0, NBA, step, 0)
        plsc.subcore_barrier()
        pltpu.sync_copy(acc.at[pl.ds(s * RPT, RPT)],
                        out_agg.at[c, pl.ds(s * RPT, RPT)])

    return pl.kernel(body, out_type=out_type, mesh=mesh, scratch_types=scratch)


def _sc_deg():
    mesh = plsc.VectorSubcoreMesh(core_axis_name="c", subcore_axis_name="s")
    out_type = jax.ShapeDtypeStruct((NC, NP), jnp.float32)
    scratch = [
        pltpu.VMEM((EPW,), jnp.int32),
        pltpu.VMEM((B,), jnp.float32),
        pltpu.VMEM_SHARED((NP,), jnp.float32),
    ]

    def body(dsts, z_deg, ones_h, out_deg, dstv, onesv, dacc):
        c = lax.axis_index("c")
        s = lax.axis_index("s")
        wid = c * NS + s
        pltpu.sync_copy(z_deg, dacc.at[pl.ds(s * RPT, RPT)])
        pltpu.sync_copy(ones_h, onesv)
        pltpu.sync_copy(dsts.at[pl.ds(wid * EPW, EPW)], dstv)
        plsc.subcore_barrier()

        def step(j, carry):
            pltpu.sync_copy(onesv, dacc.at[dstv.at[pl.ds(j * B, B)]], add=True)
            return carry

        lax.fori_loop(0, NB, step, 0)
        plsc.subcore_barrier()
        pltpu.sync_copy(dacc.at[pl.ds(s * RPT, RPT)],
                        out_deg.at[c, pl.ds(s * RPT, RPT)])

    return pl.kernel(body, out_type=out_type, mesh=mesh, scratch_types=scratch)


_AGG = _sc_agg()
_DEG = _sc_deg()


def _dense(aggp, degp, xin, WlT, bl2d, WrT, relu):
    R = 1000

    def body(aggp_ref, degp_ref, x_ref, wl_ref, bl_ref, wr_ref, out_ref):
        agg = aggp_ref[0] + aggp_ref[1]
        deg = degp_ref[0] + degp_ref[1]
        mean = agg / jnp.maximum(deg, 1.0)
        h = (jnp.dot(mean, wl_ref[...], preferred_element_type=jnp.float32)
             + bl_ref[...]
             + jnp.dot(x_ref[...], wr_ref[...], preferred_element_type=jnp.float32))
        if relu:
            h = jnp.maximum(h, 0.0)
        out_ref[...] = h

    return pl.pallas_call(
        body,
        grid=(N // R,),
        in_specs=[
            pl.BlockSpec((NC, R, D), lambda i: (0, i, 0)),
            pl.BlockSpec((NC, R, 1), lambda i: (0, i, 0)),
            pl.BlockSpec((R, D), lambda i: (i, 0)),
            pl.BlockSpec((D, D), lambda i: (0, 0)),
            pl.BlockSpec((1, D), lambda i: (0, 0)),
            pl.BlockSpec((D, D), lambda i: (0, 0)),
        ],
        out_specs=pl.BlockSpec((R, D), lambda i: (i, 0)),
        out_shape=jax.ShapeDtypeStruct((N, D), jnp.float32),
    )(aggp, degp, xin, WlT, bl2d, WrT)


def kernel(x, edge_index, Wl1, bl1, Wr1, Wl2, bl2, Wr2):
    npad = EP - E
    src = jnp.concatenate(
        [edge_index[0].astype(jnp.int32), jnp.zeros((npad,), jnp.int32)]
    )
    dst = jnp.concatenate(
        [edge_index[1].astype(jnp.int32), jnp.full((npad,), N, jnp.int32)]
    )
    z_row = jnp.zeros((RPT, D), jnp.float32)
    z_deg = jnp.zeros((RPT,), jnp.float32)
    ones_h = jnp.ones((B,), jnp.float32)

    degp = _DEG(dst, z_deg, ones_h).reshape(NC, NP, 1)
    aggp1 = _AGG(x, src, dst, z_row)
    h = _dense(aggp1, degp, x, Wl1.T, bl1.reshape(1, D), Wr1.T, relu=True)
    aggp2 = _AGG(h, src, dst, z_row)
    out = _dense(aggp2, degp, h, Wl2.T, bl2.reshape(1, D), Wr2.T, relu=False)
    return out


# speedup vs baseline: 3.5685x; 1.0832x over previous
import jax
import jax.numpy as jnp
from jax import lax
from jax.experimental import pallas as pl
from jax.experimental.pallas import tpu as pltpu
from jax.experimental.pallas import tpu_sc as plsc

N = 10000
E = 320000
D = 128
NC = 2
NS = 16
NW = NC * NS
NP = 10240
EPW = 10240
EP = NW * EPW
BE = 160
NBA = EPW // BE
B = 128
NB = EPW // B
RPT = NP // NS
DW = 16


def _sc_agg():
    mesh = plsc.VectorSubcoreMesh(core_axis_name="c", subcore_axis_name="s")
    out_type = (jax.ShapeDtypeStruct((NP, D), jnp.float32),
                jax.ShapeDtypeStruct((NP, D), jnp.float32))
    scratch = [
        pltpu.VMEM((EPW,), jnp.int32),
        pltpu.VMEM((EPW,), jnp.int32),
        pltpu.VMEM((BE, D), jnp.float32),
        pltpu.VMEM_SHARED((NP, D), jnp.float32),
    ]

    def body(x_hbm, srcs, dsts, z_row, out_a, out_b, srcv, dstv, rows, acc):
        c = lax.axis_index("c")
        s = lax.axis_index("s")
        wid = c * NS + s
        pltpu.sync_copy(z_row, acc.at[pl.ds(s * RPT, RPT)])
        pltpu.sync_copy(srcs.at[pl.ds(wid * EPW, EPW)], srcv)
        pltpu.sync_copy(dsts.at[pl.ds(wid * EPW, EPW)], dstv)
        plsc.subcore_barrier()

        def step(j, carry):
            idx = pl.ds(j * BE, BE)
            pltpu.sync_copy(x_hbm.at[srcv.at[idx]], rows)
            pltpu.sync_copy(rows, acc.at[dstv.at[idx]], add=True)
            return carry

        lax.fori_loop(0, NBA, step, 0)
        plsc.subcore_barrier()

        @pl.when(c == 0)
        def _():
            pltpu.sync_copy(acc.at[pl.ds(s * RPT, RPT)],
                            out_a.at[pl.ds(s * RPT, RPT)])

        @pl.when(c == 1)
        def _():
            pltpu.sync_copy(acc.at[pl.ds(s * RPT, RPT)],
                            out_b.at[pl.ds(s * RPT, RPT)])

    return pl.kernel(body, out_type=out_type, mesh=mesh, scratch_types=scratch)


def _sc_deg():
    mesh = plsc.VectorSubcoreMesh(core_axis_name="c", subcore_axis_name="s")
    out_type = jax.ShapeDtypeStruct((NC, NP), jnp.float32)
    scratch = [
        pltpu.VMEM((EPW,), jnp.int32),
        pltpu.VMEM((B,), jnp.float32),
        pltpu.VMEM_SHARED((NP,), jnp.float32),
    ]

    def body(dsts, z_deg, ones_h, out_deg, dstv, onesv, dacc):
        c = lax.axis_index("c")
        s = lax.axis_index("s")
        wid = c * NS + s
        pltpu.sync_copy(z_deg, dacc.at[pl.ds(s * RPT, RPT)])
        pltpu.sync_copy(ones_h, onesv)
        pltpu.sync_copy(dsts.at[pl.ds(wid * EPW, EPW)], dstv)
        plsc.subcore_barrier()

        def step(j, carry):
            pltpu.sync_copy(onesv, dacc.at[dstv.at[pl.ds(j * B, B)]], add=True)
            return carry

        lax.fori_loop(0, NB, step, 0)
        plsc.subcore_barrier()
        pltpu.sync_copy(dacc.at[pl.ds(s * RPT, RPT)],
                        out_deg.at[c, pl.ds(s * RPT, RPT)])

    return pl.kernel(body, out_type=out_type, mesh=mesh, scratch_types=scratch)


_AGG = _sc_agg()
_DEG = _sc_deg()


def _dense(agg_a, agg_b, degp, xin, WlT, bl2d, WrT, relu):
    R = 1000

    def body(agga_ref, aggb_ref, degp_ref, x_ref, wl_ref, bl_ref, wr_ref, out_ref):
        agg = agga_ref[...] + aggb_ref[...]
        deg = degp_ref[0] + degp_ref[1]
        mean = agg / jnp.maximum(deg, 1.0)
        h = (jnp.dot(mean, wl_ref[...], preferred_element_type=jnp.float32)
             + bl_ref[...]
             + jnp.dot(x_ref[...], wr_ref[...], preferred_element_type=jnp.float32))
        if relu:
            h = jnp.maximum(h, 0.0)
        out_ref[...] = h

    return pl.pallas_call(
        body,
        grid=(N // R,),
        in_specs=[
            pl.BlockSpec((R, D), lambda i: (i, 0)),
            pl.BlockSpec((R, D), lambda i: (i, 0)),
            pl.BlockSpec((NC, R, 1), lambda i: (0, i, 0)),
            pl.BlockSpec((R, D), lambda i: (i, 0)),
            pl.BlockSpec((D, D), lambda i: (0, 0)),
            pl.BlockSpec((1, D), lambda i: (0, 0)),
            pl.BlockSpec((D, D), lambda i: (0, 0)),
        ],
        out_specs=pl.BlockSpec((R, D), lambda i: (i, 0)),
        out_shape=jax.ShapeDtypeStruct((N, D), jnp.float32),
    )(agg_a, agg_b, degp, xin, WlT, bl2d, WrT)


def kernel(x, edge_index, Wl1, bl1, Wr1, Wl2, bl2, Wr2):
    npad = EP - E
    src = jnp.concatenate(
        [edge_index[0].astype(jnp.int32), jnp.zeros((npad,), jnp.int32)]
    )
    dst = jnp.concatenate(
        [edge_index[1].astype(jnp.int32), jnp.full((npad,), N, jnp.int32)]
    )
    z_row = jnp.zeros((RPT, D), jnp.float32)
    z_deg = jnp.zeros((RPT,), jnp.float32)
    ones_h = jnp.ones((B,), jnp.float32)

    degp = _DEG(dst, z_deg, ones_h).reshape(NC, NP, 1)
    a1, b1 = _AGG(x, src, dst, z_row)
    h = _dense(a1, b1, degp, x, Wl1.T, bl1.reshape(1, D), Wr1.T, relu=True)
    a2, b2 = _AGG(h, src, dst, z_row)
    out = _dense(a2, b2, degp, h, Wl2.T, bl2.reshape(1, D), Wr2.T, relu=False)
    return out
